# Initial kernel scaffold; baseline (speedup 1.0000x reference)
import jax
import jax.numpy as jnp
from jax.experimental import pallas as pl


def kernel(attention):
    raise NotImplementedError("write your pallas kernel here")


# speedup vs baseline: 31.0639x; 31.0639x over previous
import jax
import jax.numpy as jnp
from jax import lax
from jax.experimental import pallas as pl
from jax.experimental.pallas import tpu as pltpu

_K_BY_HEAD = (10, 20, 40, 500)
_ROW_BLOCK = 256


def _topk_softmax_body(x_ref, o_ref):
    h = pl.program_id(1)
    x = x_ref[0, 0]
    r, n = x.shape
    k = jnp.where(
        h == 0, _K_BY_HEAD[0],
        jnp.where(h == 1, _K_BY_HEAD[1],
                  jnp.where(h == 2, _K_BY_HEAD[2], _K_BY_HEAD[3])))
    k = jnp.minimum(k, n).astype(jnp.int32)

    b = lax.bitcast_convert_type(x, jnp.int32)
    key = jnp.where(b >= 0, b, b ^ jnp.int32(0x7FFFFFFF))

    def count_ge(cand):
        return jnp.sum((key >= cand).astype(jnp.int32), axis=-1, keepdims=True)

    t = jnp.full((r, 1), jnp.int32(-2147483648))
    cand0 = jnp.zeros((r, 1), jnp.int32)
    t = jnp.where(count_ge(cand0) >= k, cand0, t)

    def step(i, t):
        cand = t + (jnp.int32(1) << (jnp.int32(30) - i))
        return jnp.where(count_ge(cand) >= k, cand, t)

    t = lax.fori_loop(0, 31, step, t, unroll=True)

    m = jnp.max(x, axis=-1, keepdims=True)
    e = jnp.exp(x - m)
    sel = key >= t
    z = jnp.sum(jnp.where(sel, e, 0.0), axis=-1, keepdims=True)
    o_ref[0, 0] = jnp.where(sel, e / z, 0.0)


def kernel(attention):
    bsz, heads, n, _ = attention.shape
    r = min(_ROW_BLOCK, n)
    grid = (bsz, heads, n // r)
    return pl.pallas_call(
        _topk_softmax_body,
        grid=grid,
        in_specs=[pl.BlockSpec((1, 1, r, n), lambda b, h, i: (b, h, i, 0))],
        out_specs=pl.BlockSpec((1, 1, r, n), lambda b, h, i: (b, h, i, 0)),
        out_shape=jax.ShapeDtypeStruct(attention.shape, attention.dtype),
        compiler_params=pltpu.CompilerParams(
            dimension_semantics=("parallel", "parallel", "arbitrary")),
    )(attention)
